# baseline (speedup 1.0000x reference)
import functools
import jax
import jax.numpy as jnp
from jax import lax
from jax.experimental import pallas as pl
from jax.experimental.pallas import tpu as pltpu
from jax.experimental.pallas import tpu_sc as plsc

D_MODEL = 768
VOCAB = 100000
TOKENS = 4 * 2048
NUM_WORKERS = 32
CH = 16
NBUF = 8
S_SC = 6144
TPW = S_SC // NUM_WORKERS
NCH = TPW // CH
T_TC = TOKENS - S_SC
K_TC = 4


def _sc_body(x_hbm, wt_hbm, out_hbm, x_vm, *rest):
    cid = lax.axis_index("c")
    sid = lax.axis_index("s")
    wid = sid * 2 + cid
    base = wid * TPW
    pltpu.sync_copy(x_hbm.at[wid], x_vm)

    bufs = rest[:NBUF]
    gsems = rest[NBUF:2 * NBUF]
    wsems = rest[2 * NBUF:]
    gathers = [None] * NCH
    writes = [None] * NCH
    for c in range(NBUF):
        gathers[c] = pltpu.async_copy(
            wt_hbm.at[x_vm.at[c]], bufs[c], gsems[c]
        )
    for c in range(NCH):
        k = c % NBUF
        gathers[c].wait()
        writes[c] = pltpu.async_copy(
            bufs[k], out_hbm.at[pl.ds(base + c * CH, CH)], wsems[k]
        )
        if c + NBUF < NCH:
            writes[c].wait()
            gathers[c + NBUF] = pltpu.async_copy(
                wt_hbm.at[x_vm.at[c + NBUF]], bufs[k], gsems[k]
            )
    for c in range(NCH - NBUF, NCH):
        if writes[c] is not None:
            writes[c].wait()


def _tc_body(x_sm, wt_hbm, out_vm, stage, *sems):
    li = lax.broadcasted_iota(jnp.int32, (8, 64), 1)
    ri = lax.broadcasted_iota(jnp.int32, (8, 64), 0)

    def issue_group(g, slot):
        for j in range(8):
            idx = x_sm[g * 8 + j]
            pltpu.make_async_copy(
                wt_hbm.at[pl.ds(idx // 8, 1)],
                stage.at[pl.ds(slot * 8 + j, 1)],
                sems[slot],
            ).start()

    def process_group(g, slot):
        for j in range(8):
            pltpu.make_async_copy(
                wt_hbm.at[pl.ds(0, 1)],
                stage.at[pl.ds(slot * 8 + j, 1)],
                sems[slot],
            ).wait()
        sel = jnp.zeros((8, 64), jnp.float32)
        for j in range(8):
            tgt = j * 8 + x_sm[g * 8 + j] % 8
            sel = jnp.where((li == tgt) & (ri == j), 1.0, sel)
        big = stage[pl.ds(slot * 8, 8)].reshape(64, D_MODEL)
        rows = jnp.dot(sel, big, preferred_element_type=jnp.float32)
        out_vm[pl.ds(g, 1)] = rows.reshape(1, 8, D_MODEL)

    n_groups = T_TC // 8
    for s in range(K_TC):
        issue_group(s, s)

    def macro(m, _):
        g0 = m * K_TC
        for k in range(K_TC):
            process_group(g0 + k, k)
            issue_group(g0 + k + K_TC, k)
        return 0

    lax.fori_loop(0, n_groups // K_TC - 1, macro, 0)
    for k in range(K_TC):
        process_group(n_groups - K_TC + k, k)


@jax.jit
def _embed(xf_sc, x_tc, wt):
    mesh = plsc.VectorSubcoreMesh(core_axis_name="c", subcore_axis_name="s")
    sc = functools.partial(
        pl.kernel,
        mesh=mesh,
        out_type=jax.ShapeDtypeStruct((TOKENS, D_MODEL), jnp.float32),
        scratch_types=(
            [pltpu.VMEM((NCH, CH), jnp.int32)]
            + [pltpu.VMEM((CH, D_MODEL), jnp.float32)] * NBUF
            + [pltpu.SemaphoreType.DMA] * (2 * NBUF)
        ),
    )(_sc_body)
    out_sc = sc(xf_sc, wt)

    wt3 = wt.reshape(VOCAB // 8, 8, D_MODEL)
    out_tc = pl.pallas_call(
        _tc_body,
        out_shape=jax.ShapeDtypeStruct((T_TC // 8, 8, D_MODEL), jnp.float32),
        in_specs=[
            pl.BlockSpec(memory_space=pltpu.SMEM),
            pl.BlockSpec(memory_space=pl.ANY),
        ],
        out_specs=pl.BlockSpec(memory_space=pltpu.VMEM),
        scratch_shapes=(
            [pltpu.VMEM((K_TC * 8, 8, D_MODEL), jnp.float32)]
            + [pltpu.SemaphoreType.DMA] * K_TC
        ),
    )(x_tc, wt3)

    return lax.dynamic_update_slice(
        out_sc, out_tc.reshape(T_TC, D_MODEL), (S_SC, 0)
    )


def kernel(x, W_E):
    xf = x.reshape(TOKENS).astype(jnp.int32)
    xf_sc = xf[:S_SC].reshape(NUM_WORKERS, NCH, CH)
    x_tc = xf[S_SC:]
    wt = W_E.T
    out = _embed(xf_sc, x_tc, wt)
    return out.reshape(4, 2048, D_MODEL)


# speedup vs baseline: 2.4434x; 2.4434x over previous
import functools
import jax
import jax.numpy as jnp
from jax import lax
from jax.experimental import pallas as pl
from jax.experimental.pallas import tpu as pltpu
from jax.experimental.pallas import tpu_sc as plsc

D_MODEL = 768
VOCAB = 100000
TOKENS = 4 * 2048
NUM_CORES = 1
NUM_WORKERS = 16 * NUM_CORES
TPW = TOKENS // NUM_WORKERS
CH = 16
NCH = TPW // CH
NBUF = 8


def _body(x_hbm, wt_hbm, out_hbm, x_vm, *rest):
    cid = lax.axis_index("c")
    sid = lax.axis_index("s")
    wid = sid * NUM_CORES + cid
    base = wid * TPW
    pltpu.sync_copy(x_hbm.at[wid], x_vm)

    bufs = rest[:NBUF]
    gsems = rest[NBUF:2 * NBUF]
    wsems = rest[2 * NBUF:]
    gathers = [None] * NCH
    writes = [None] * NCH
    for c in range(NBUF):
        gathers[c] = pltpu.async_copy(
            wt_hbm.at[x_vm.at[c]], bufs[c], gsems[c]
        )
    for c in range(NCH):
        k = c % NBUF
        gathers[c].wait()
        writes[c] = pltpu.async_copy(
            bufs[k], out_hbm.at[pl.ds(base + c * CH, CH)], wsems[k]
        )
        if c + NBUF < NCH:
            writes[c].wait()
            gathers[c + NBUF] = pltpu.async_copy(
                wt_hbm.at[x_vm.at[c + NBUF]], bufs[k], gsems[k]
            )
    for c in range(NCH - NBUF, NCH):
        if writes[c] is not None:
            writes[c].wait()


@jax.jit
def _embed(xf, wt):
    mesh = plsc.VectorSubcoreMesh(
        core_axis_name="c", subcore_axis_name="s", num_cores=NUM_CORES
    )
    f = functools.partial(
        pl.kernel,
        mesh=mesh,
        out_type=jax.ShapeDtypeStruct((TOKENS, D_MODEL), jnp.float32),
        scratch_types=(
            [pltpu.VMEM((NCH, CH), jnp.int32)]
            + [pltpu.VMEM((CH, D_MODEL), jnp.float32)] * NBUF
            + [pltpu.SemaphoreType.DMA] * (2 * NBUF)
        ),
    )(_body)
    return f(xf, wt)


def kernel(x, W_E):
    xf = x.reshape(NUM_WORKERS, NCH, CH).astype(jnp.int32)
    wt = W_E.T
    out = _embed(xf, wt)
    return out.reshape(4, 2048, D_MODEL)


# speedup vs baseline: 2.6699x; 1.0927x over previous
import functools
import jax
import jax.numpy as jnp
from jax import lax
from jax.experimental import pallas as pl
from jax.experimental.pallas import tpu as pltpu
from jax.experimental.pallas import tpu_sc as plsc

D_MODEL = 768
VOCAB = 100000
TOKENS = 4 * 2048
NUM_CORES = 2
NUM_WORKERS = 16 * NUM_CORES
TPW = TOKENS // NUM_WORKERS
CH = 16
NCH = TPW // CH
NBUF = 8


def _body(x_hbm, wt_hbm, out_hbm, x_vm, *rest):
    cid = lax.axis_index("c")
    sid = lax.axis_index("s")
    wid = sid * NUM_CORES + cid
    base = wid * TPW
    pltpu.sync_copy(x_hbm.at[wid], x_vm)

    bufs = rest[:NBUF]
    gsems = rest[NBUF:2 * NBUF]
    wsems = rest[2 * NBUF:]
    gathers = [None] * NCH
    writes = [None] * NCH
    for c in range(NBUF):
        gathers[c] = pltpu.async_copy(
            wt_hbm.at[x_vm.at[c]], bufs[c], gsems[c]
        )
    for c in range(NCH):
        k = c % NBUF
        gathers[c].wait()
        writes[c] = pltpu.async_copy(
            bufs[k], out_hbm.at[pl.ds(base + c * CH, CH)], wsems[k]
        )
        if c + NBUF < NCH:
            writes[c].wait()
            gathers[c + NBUF] = pltpu.async_copy(
                wt_hbm.at[x_vm.at[c + NBUF]], bufs[k], gsems[k]
            )
    for c in range(NCH - NBUF, NCH):
        if writes[c] is not None:
            writes[c].wait()


@jax.jit
def _embed(xf, wt):
    mesh = plsc.VectorSubcoreMesh(
        core_axis_name="c", subcore_axis_name="s", num_cores=NUM_CORES
    )
    f = functools.partial(
        pl.kernel,
        mesh=mesh,
        out_type=jax.ShapeDtypeStruct((TOKENS, D_MODEL), jnp.float32),
        scratch_types=(
            [pltpu.VMEM((NCH, CH), jnp.int32)]
            + [pltpu.VMEM((CH, D_MODEL), jnp.float32)] * NBUF
            + [pltpu.SemaphoreType.DMA] * (2 * NBUF)
        ),
    )(_body)
    return f(xf, wt)


def kernel(x, W_E):
    xf = x.reshape(NUM_WORKERS, NCH, CH).astype(jnp.int32)
    wt = W_E.T
    out = _embed(xf, wt)
    return out.reshape(4, 2048, D_MODEL)
